# baseline (speedup 1.0000x reference)
import functools

import jax
import jax.numpy as jnp
from jax import lax
from jax.experimental import pallas as pl
from jax.experimental.pallas import tpu as pltpu
from jax.experimental.pallas import tpu_sc as plsc

NUM_ROWS = 1000000
LATENT_DIM = 16
BATCH = 16384
REG_BIAS = 0.00013
REG_LATENT = 0.00018

NC, NS, L = 2, 16, 16
NW = NC * NS
BPW = BATCH // NW
CHUNK = 128
NCHUNK = BPW // CHUNK
NBLK = BPW // L

_mesh = plsc.VectorSubcoreMesh(
    core_axis_name="c", subcore_axis_name="s", num_cores=NC, num_subcores=NS
)


@functools.partial(
    pl.kernel,
    out_type=[
        jax.ShapeDtypeStruct((BATCH,), jnp.float32),
        jax.ShapeDtypeStruct((NW * L,), jnp.float32),
        jax.ShapeDtypeStruct((NW * L,), jnp.float32),
        jax.ShapeDtypeStruct((NW * L,), jnp.float32),
    ],
    mesh=_mesh,
    scratch_types=[
        pltpu.VMEM((NCHUNK, L, CHUNK), jnp.int32),
        pltpu.VMEM((NCHUNK, L, CHUNK), jnp.int32),
        pltpu.VMEM((NCHUNK, L, CHUNK), jnp.int32),
        pltpu.VMEM((NCHUNK, CHUNK), jnp.int32),
        pltpu.VMEM((NCHUNK, CHUNK), jnp.int32),
        pltpu.VMEM((L, BPW), jnp.float32),
        pltpu.VMEM((L, BPW), jnp.float32),
        pltpu.VMEM((L, BPW), jnp.float32),
        pltpu.VMEM((BPW,), jnp.float32),
        pltpu.VMEM((BPW,), jnp.float32),
        pltpu.VMEM((BPW,), jnp.float32),
        pltpu.VMEM((L,), jnp.float32),
        pltpu.VMEM((L,), jnp.float32),
        pltpu.VMEM((L,), jnp.float32),
        pltpu.SemaphoreType.DMA,
    ],
)
def _sc_scores(uflat, iflat, ib, uifl, pifl, nifl, pib, nib,
               x_out, se_out, sp_out, sn_out,
               uidx, pidx, nidx, pbx, nbx, ufeat, pfeat, nfeat, pb, nb,
               xv, sev, spv, snv, sem):
    wid = lax.axis_index("s") * NC + lax.axis_index("c")
    base = wid * BPW

    pltpu.sync_copy(uifl.at[wid], uidx)
    pltpu.sync_copy(pifl.at[wid], pidx)
    pltpu.sync_copy(nifl.at[wid], nidx)
    pltpu.sync_copy(pib.at[wid], pbx)
    pltpu.sync_copy(nib.at[wid], nbx)

    copies = []
    for c in range(NCHUNK):
        sl = pl.ds(c * CHUNK, CHUNK)
        for j in range(L):
            copies.append(
                pltpu.async_copy(uflat.at[uidx.at[c, j]], ufeat.at[j, sl], sem))
            copies.append(
                pltpu.async_copy(iflat.at[pidx.at[c, j]], pfeat.at[j, sl], sem))
            copies.append(
                pltpu.async_copy(iflat.at[nidx.at[c, j]], nfeat.at[j, sl], sem))
        copies.append(pltpu.async_copy(ib.at[pbx.at[c]], pb.at[sl], sem))
        copies.append(pltpu.async_copy(ib.at[nbx.at[c]], nb.at[sl], sem))
    for cp in copies:
        cp.wait()

    fzero = jnp.zeros((L,), jnp.float32)
    se_acc = fzero
    sp_acc = fzero
    sn_acc = fzero
    for blk in range(NBLK):
        sl = pl.ds(blk * L, L)
        pbv = pb[sl]
        nbv = nb[sl]
        xs = pbv - nbv
        for j in range(L):
            u = ufeat[j, sl]
            p = pfeat[j, sl]
            n = nfeat[j, sl]
            xs = xs + u * (p - n)
            se_acc = se_acc + u * u + p * p + n * n
        xv[sl] = xs
        sp_acc = sp_acc + pbv * pbv
        sn_acc = sn_acc + nbv * nbv

    sev[...] = se_acc
    spv[...] = sp_acc
    snv[...] = sn_acc

    pltpu.sync_copy(xv, x_out.at[pl.ds(base, BPW)])
    pltpu.sync_copy(sev, se_out.at[pl.ds(wid * L, L)])
    pltpu.sync_copy(spv, sp_out.at[pl.ds(wid * L, L)])
    pltpu.sync_copy(snv, sn_out.at[pl.ds(wid * L, L)])


def _tc_loss_body(x_ref, se_ref, sp_ref, sn_ref, o_ref):
    x = x_ref[...]
    s = 1.0 / (1.0 + jnp.exp(-x)) + 1e-10
    loss = -jnp.sum(jnp.log(s)) / BATCH
    reg = REG_BIAS * (
        jnp.sqrt(jnp.sum(sp_ref[...])) + jnp.sqrt(jnp.sum(sn_ref[...]))
    ) * 0.5
    reg = reg + REG_LATENT * jnp.sum(se_ref[...])
    o_ref[...] = jnp.broadcast_to(loss + reg, (1, 1))


_tc_loss = pl.pallas_call(
    _tc_loss_body,
    out_shape=jax.ShapeDtypeStruct((1, 1), jnp.float32),
)


def _flat_feature_indices(idx):
    shifted = idx[None, :] + (jnp.arange(L, dtype=jnp.int32) * NUM_ROWS)[:, None]
    return shifted.reshape(L, NW, NCHUNK, CHUNK).transpose(1, 2, 0, 3)


def kernel(user_factors, item_factors, item_bias,
           user_indices, pos_item_indices, neg_item_indices):
    uflat = user_factors.T.reshape(-1)
    iflat = item_factors.T.reshape(-1)
    ibf = item_bias.reshape(-1)
    ui = user_indices.astype(jnp.int32)
    pi = pos_item_indices.astype(jnp.int32)
    ni = neg_item_indices.astype(jnp.int32)
    x, se, sp, sn = _sc_scores(
        uflat, iflat, ibf,
        _flat_feature_indices(ui), _flat_feature_indices(pi),
        _flat_feature_indices(ni),
        pi.reshape(NW, NCHUNK, CHUNK), ni.reshape(NW, NCHUNK, CHUNK),
    )
    out = _tc_loss(x.reshape(BATCH // 128, 128), se.reshape(4, 128),
                   sp.reshape(4, 128), sn.reshape(4, 128))
    return out[0, 0]


# speedup vs baseline: 1.0024x; 1.0024x over previous
import functools

import jax
import jax.numpy as jnp
from jax import lax
from jax.experimental import pallas as pl
from jax.experimental.pallas import tpu as pltpu
from jax.experimental.pallas import tpu_sc as plsc

NUM_ROWS = 1000000
LATENT_DIM = 16
BATCH = 16384
REG_BIAS = 0.00013
REG_LATENT = 0.00018

NC, NS, L = 2, 16, 16
NW = NC * NS
BPW = BATCH // NW
CHUNK = 128
NCHUNK = BPW // CHUNK
NBLK = BPW // L

_mesh = plsc.VectorSubcoreMesh(
    core_axis_name="c", subcore_axis_name="s", num_cores=NC, num_subcores=NS
)


@functools.partial(
    pl.kernel,
    out_type=[
        jax.ShapeDtypeStruct((BATCH,), jnp.float32),
        jax.ShapeDtypeStruct((NW * L,), jnp.float32),
        jax.ShapeDtypeStruct((NW * L,), jnp.float32),
        jax.ShapeDtypeStruct((NW * L,), jnp.float32),
    ],
    mesh=_mesh,
    compiler_params=pltpu.CompilerParams(use_tc_tiling_on_sc=False),
    scratch_types=[
        pltpu.VMEM((NCHUNK, CHUNK), jnp.int32),
        pltpu.VMEM((NCHUNK, CHUNK), jnp.int32),
        pltpu.VMEM((NCHUNK, CHUNK), jnp.int32),
        pltpu.VMEM((L, BPW), jnp.float32),
        pltpu.VMEM((L, BPW), jnp.float32),
        pltpu.VMEM((L, BPW), jnp.float32),
        pltpu.VMEM((BPW,), jnp.float32),
        pltpu.VMEM((BPW,), jnp.float32),
        pltpu.VMEM((BPW,), jnp.float32),
        pltpu.VMEM((L,), jnp.float32),
        pltpu.VMEM((L,), jnp.float32),
        pltpu.VMEM((L,), jnp.float32),
        pltpu.SemaphoreType.DMA,
    ],
)
def _sc_scores(uft, ift, ib, ui, pi, ni,
               x_out, se_out, sp_out, sn_out,
               uidx, pidx, nidx, ufeat, pfeat, nfeat, pb, nb,
               xv, sev, spv, snv, sem):
    wid = lax.axis_index("s") * NC + lax.axis_index("c")
    base = wid * BPW

    pltpu.sync_copy(ui.at[wid], uidx)
    pltpu.sync_copy(pi.at[wid], pidx)
    pltpu.sync_copy(ni.at[wid], nidx)

    copies = []
    for c in range(NCHUNK):
        sl = pl.ds(c * CHUNK, CHUNK)
        for j in range(L):
            copies.append(
                pltpu.async_copy(uft.at[j].at[uidx.at[c]], ufeat.at[j, sl], sem))
            copies.append(
                pltpu.async_copy(ift.at[j].at[pidx.at[c]], pfeat.at[j, sl], sem))
            copies.append(
                pltpu.async_copy(ift.at[j].at[nidx.at[c]], nfeat.at[j, sl], sem))
        copies.append(pltpu.async_copy(ib.at[pidx.at[c]], pb.at[sl], sem))
        copies.append(pltpu.async_copy(ib.at[nidx.at[c]], nb.at[sl], sem))
    for cp in copies:
        cp.wait()

    fzero = jnp.zeros((L,), jnp.float32)
    se_acc = fzero
    sp_acc = fzero
    sn_acc = fzero
    for blk in range(NBLK):
        sl = pl.ds(blk * L, L)
        pbv = pb[sl]
        nbv = nb[sl]
        xs = pbv - nbv
        for j in range(L):
            u = ufeat[j, sl]
            p = pfeat[j, sl]
            n = nfeat[j, sl]
            xs = xs + u * (p - n)
            se_acc = se_acc + u * u + p * p + n * n
        xv[sl] = xs
        sp_acc = sp_acc + pbv * pbv
        sn_acc = sn_acc + nbv * nbv

    sev[...] = se_acc
    spv[...] = sp_acc
    snv[...] = sn_acc

    pltpu.sync_copy(xv, x_out.at[pl.ds(base, BPW)])
    pltpu.sync_copy(sev, se_out.at[pl.ds(wid * L, L)])
    pltpu.sync_copy(spv, sp_out.at[pl.ds(wid * L, L)])
    pltpu.sync_copy(snv, sn_out.at[pl.ds(wid * L, L)])


def _tc_loss_body(x_ref, se_ref, sp_ref, sn_ref, o_ref):
    x = x_ref[...]
    s = 1.0 / (1.0 + jnp.exp(-x)) + 1e-10
    loss = -jnp.sum(jnp.log(s)) / BATCH
    reg = REG_BIAS * (
        jnp.sqrt(jnp.sum(sp_ref[...])) + jnp.sqrt(jnp.sum(sn_ref[...]))
    ) * 0.5
    reg = reg + REG_LATENT * jnp.sum(se_ref[...])
    o_ref[...] = jnp.broadcast_to(loss + reg, (1, 1))


_tc_loss = pl.pallas_call(
    _tc_loss_body,
    out_shape=jax.ShapeDtypeStruct((1, 1), jnp.float32),
)


def kernel(user_factors, item_factors, item_bias,
           user_indices, pos_item_indices, neg_item_indices):
    ui = user_indices.astype(jnp.int32).reshape(NW, NCHUNK, CHUNK)
    pi = pos_item_indices.astype(jnp.int32).reshape(NW, NCHUNK, CHUNK)
    ni = neg_item_indices.astype(jnp.int32).reshape(NW, NCHUNK, CHUNK)
    x, se, sp, sn = _sc_scores(user_factors.T, item_factors.T,
                               item_bias.reshape(-1), ui, pi, ni)
    out = _tc_loss(x.reshape(BATCH // 128, 128), se.reshape(4, 128),
                   sp.reshape(4, 128), sn.reshape(4, 128))
    return out[0, 0]


# speedup vs baseline: 13.9277x; 13.8946x over previous
import functools

import jax
import jax.numpy as jnp
from jax import lax
from jax.experimental import pallas as pl
from jax.experimental.pallas import tpu as pltpu
from jax.experimental.pallas import tpu_sc as plsc

NUM_ROWS = 1000000
LATENT_DIM = 16
BATCH = 16384
REG_BIAS = 0.00013
REG_LATENT = 0.00018

NC, NS, L = 2, 16, 16
NW = NC * NS
BPW = BATCH // NW
CHUNK = 128
NCHUNK = BPW // CHUNK
NBLK = BPW // L

NFULL = NUM_ROWS // 128
UBLK = NFULL + 1
TAIL0 = NUM_ROWS - 128
GROUP = 16

_mesh = plsc.VectorSubcoreMesh(
    core_axis_name="c", subcore_axis_name="s", num_cores=NC, num_subcores=NS
)


@functools.partial(
    pl.kernel,
    out_type=[
        jax.ShapeDtypeStruct((UBLK, L, 128), jnp.float32),
        jax.ShapeDtypeStruct((UBLK, L, 128), jnp.float32),
    ],
    mesh=_mesh,
    scratch_types=[
        pltpu.VMEM((GROUP, 8, 128), jnp.float32),
        pltpu.VMEM((GROUP, 8, 128), jnp.float32),
        pltpu.SemaphoreType.DMA,
        pltpu.SemaphoreType.DMA,
    ],
)
def _sc_detile(uft, ift, utail, itail, uout, iout, ring0, ring8, semr, semw):
    wid = lax.axis_index("s") * NC + lax.axis_index("c")
    nk = (NFULL - 1 - wid) // NW + 1
    ngrp = (nk - 1) // GROUP + 1

    def run_table(src, dst):
        dummy = src.at[pl.ds(0, 8), pl.ds(0, 128)]

        def group(g, carry):
            k0 = g * GROUP

            def fire_read(t, c):
                @pl.when(k0 + t < nk)
                def _():
                    c0 = pl.multiple_of((wid + (k0 + t) * NW) * 128, 128)
                    pltpu.async_copy(
                        src.at[pl.ds(0, 8), pl.ds(c0, 128)], ring0.at[t], semr)
                    pltpu.async_copy(
                        src.at[pl.ds(8, 8), pl.ds(c0, 128)], ring8.at[t], semr)
                return c

            def drain_r(t, c):
                @pl.when(k0 + t < nk)
                def _():
                    pltpu.make_async_copy(dummy, ring0.at[0], semr).wait()
                    pltpu.make_async_copy(dummy, ring8.at[0], semr).wait()
                return c

            def drain_w(t, c):
                @pl.when(k0 + t < nk)
                def _():
                    pltpu.make_async_copy(dummy, ring0.at[0], semw).wait()
                    pltpu.make_async_copy(dummy, ring8.at[0], semw).wait()
                return c

            def fire_write(t, c):
                @pl.when(k0 + t < nk)
                def _():
                    u_blk = wid + (k0 + t) * NW
                    pltpu.async_copy(
                        ring0.at[t], dst.at[u_blk, pl.ds(0, 8), :], semw)
                    pltpu.async_copy(
                        ring8.at[t], dst.at[u_blk, pl.ds(8, 8), :], semw)
                return c

            lax.fori_loop(0, GROUP, fire_read, 0)
            lax.fori_loop(0, GROUP, drain_r, 0)
            lax.fori_loop(0, GROUP, fire_write, 0)
            lax.fori_loop(0, GROUP, drain_w, 0)
            return carry

        lax.fori_loop(0, ngrp, group, 0)

    run_table(uft, uout)
    run_table(ift, iout)

    @pl.when(wid == 0)
    def _():
        pltpu.sync_copy(utail.at[pl.ds(0, 8), :], ring0.at[0])
        pltpu.sync_copy(utail.at[pl.ds(8, 8), :], ring8.at[0])
        pltpu.sync_copy(ring0.at[0], uout.at[NFULL, pl.ds(0, 8), :])
        pltpu.sync_copy(ring8.at[0], uout.at[NFULL, pl.ds(8, 8), :])

    @pl.when(wid == 1)
    def _():
        pltpu.sync_copy(itail.at[pl.ds(0, 8), :], ring0.at[0])
        pltpu.sync_copy(itail.at[pl.ds(8, 8), :], ring8.at[0])
        pltpu.sync_copy(ring0.at[0], iout.at[NFULL, pl.ds(0, 8), :])
        pltpu.sync_copy(ring8.at[0], iout.at[NFULL, pl.ds(8, 8), :])


@functools.partial(
    pl.kernel,
    out_type=[
        jax.ShapeDtypeStruct((BATCH,), jnp.float32),
        jax.ShapeDtypeStruct((NW * L,), jnp.float32),
        jax.ShapeDtypeStruct((NW * L,), jnp.float32),
        jax.ShapeDtypeStruct((NW * L,), jnp.float32),
    ],
    mesh=_mesh,
    compiler_params=pltpu.CompilerParams(use_tc_tiling_on_sc=False),
    scratch_types=[
        pltpu.VMEM((NCHUNK, L, CHUNK), jnp.int32),
        pltpu.VMEM((NCHUNK, L, CHUNK), jnp.int32),
        pltpu.VMEM((NCHUNK, L, CHUNK), jnp.int32),
        pltpu.VMEM((NCHUNK, CHUNK), jnp.int32),
        pltpu.VMEM((NCHUNK, CHUNK), jnp.int32),
        pltpu.VMEM((L, BPW), jnp.float32),
        pltpu.VMEM((L, BPW), jnp.float32),
        pltpu.VMEM((L, BPW), jnp.float32),
        pltpu.VMEM((BPW,), jnp.float32),
        pltpu.VMEM((BPW,), jnp.float32),
        pltpu.VMEM((BPW,), jnp.float32),
        pltpu.VMEM((L,), jnp.float32),
        pltpu.VMEM((L,), jnp.float32),
        pltpu.VMEM((L,), jnp.float32),
        pltpu.SemaphoreType.DMA,
    ],
)
def _sc_scores(uflat, iflat, ib, uifl, pifl, nifl, pib, nib,
               x_out, se_out, sp_out, sn_out,
               uidx, pidx, nidx, pbx, nbx, ufeat, pfeat, nfeat, pb, nb,
               xv, sev, spv, snv, sem):
    wid = lax.axis_index("s") * NC + lax.axis_index("c")
    base = wid * BPW

    pltpu.sync_copy(uifl.at[wid], uidx)
    pltpu.sync_copy(pifl.at[wid], pidx)
    pltpu.sync_copy(nifl.at[wid], nidx)
    pltpu.sync_copy(pib.at[wid], pbx)
    pltpu.sync_copy(nib.at[wid], nbx)

    copies = []
    for c in range(NCHUNK):
        sl = pl.ds(c * CHUNK, CHUNK)
        for j in range(L):
            copies.append(
                pltpu.async_copy(uflat.at[uidx.at[c, j]], ufeat.at[j, sl], sem))
            copies.append(
                pltpu.async_copy(iflat.at[pidx.at[c, j]], pfeat.at[j, sl], sem))
            copies.append(
                pltpu.async_copy(iflat.at[nidx.at[c, j]], nfeat.at[j, sl], sem))
        copies.append(pltpu.async_copy(ib.at[pbx.at[c]], pb.at[sl], sem))
        copies.append(pltpu.async_copy(ib.at[nbx.at[c]], nb.at[sl], sem))
    for cp in copies:
        cp.wait()

    fzero = jnp.zeros((L,), jnp.float32)
    se_acc = fzero
    sp_acc = fzero
    sn_acc = fzero
    for blk in range(NBLK):
        sl = pl.ds(blk * L, L)
        pbv = pb[sl]
        nbv = nb[sl]
        xs = pbv - nbv
        for j in range(L):
            u = ufeat[j, sl]
            p = pfeat[j, sl]
            n = nfeat[j, sl]
            xs = xs + u * (p - n)
            se_acc = se_acc + u * u + p * p + n * n
        xv[sl] = xs
        sp_acc = sp_acc + pbv * pbv
        sn_acc = sn_acc + nbv * nbv

    sev[...] = se_acc
    spv[...] = sp_acc
    snv[...] = sn_acc

    pltpu.sync_copy(xv, x_out.at[pl.ds(base, BPW)])
    pltpu.sync_copy(sev, se_out.at[pl.ds(wid * L, L)])
    pltpu.sync_copy(spv, sp_out.at[pl.ds(wid * L, L)])
    pltpu.sync_copy(snv, sn_out.at[pl.ds(wid * L, L)])


def _tc_loss_body(x_ref, se_ref, sp_ref, sn_ref, o_ref):
    x = x_ref[...]
    s = 1.0 / (1.0 + jnp.exp(-x)) + 1e-10
    loss = -jnp.sum(jnp.log(s)) / BATCH
    reg = REG_BIAS * (
        jnp.sqrt(jnp.sum(sp_ref[...])) + jnp.sqrt(jnp.sum(sn_ref[...]))
    ) * 0.5
    reg = reg + REG_LATENT * jnp.sum(se_ref[...])
    o_ref[...] = jnp.broadcast_to(loss + reg, (1, 1))


_tc_loss = pl.pallas_call(
    _tc_loss_body,
    out_shape=jax.ShapeDtypeStruct((1, 1), jnp.float32),
)


def _flat_feature_indices(idx):
    blk = jnp.where(idx < NFULL * 128, idx // 128, NFULL)
    lane = jnp.where(idx < NFULL * 128, idx % 128, idx - TAIL0)
    base = (blk * (L * 128) + lane).astype(jnp.int32)
    shifted = base[None, :] + (jnp.arange(L, dtype=jnp.int32) * 128)[:, None]
    return shifted.reshape(L, NW, NCHUNK, CHUNK).transpose(1, 2, 0, 3)


def kernel(user_factors, item_factors, item_bias,
           user_indices, pos_item_indices, neg_item_indices):
    uft = user_factors.T
    ift = item_factors.T
    utail = uft[:, TAIL0:]
    itail = ift[:, TAIL0:]
    uout, iout = _sc_detile(uft, ift, utail, itail)
    uflat = uout.reshape(-1)
    iflat = iout.reshape(-1)
    ibf = item_bias.reshape(-1)
    ui = user_indices.astype(jnp.int32)
    pi = pos_item_indices.astype(jnp.int32)
    ni = neg_item_indices.astype(jnp.int32)
    x, se, sp, sn = _sc_scores(
        uflat, iflat, ibf,
        _flat_feature_indices(ui), _flat_feature_indices(pi),
        _flat_feature_indices(ni),
        pi.reshape(NW, NCHUNK, CHUNK), ni.reshape(NW, NCHUNK, CHUNK),
    )
    out = _tc_loss(x.reshape(BATCH // 128, 128), se.reshape(4, 128),
                   sp.reshape(4, 128), sn.reshape(4, 128))
    return out[0, 0]
